# baseline (speedup 1.0000x reference)
import functools

import jax
import jax.numpy as jnp
from jax import lax
from jax.experimental import pallas as pl
from jax.experimental.pallas import tpu as pltpu
from jax.experimental.pallas import tpu_sc as plsc

L = 16
NW = 32
G = 26
D = 256
CH = 160
NGRP = CH // L


def _sc_body(g_hbm, f_hbm, p_hbm, rt_hbm, rm_hbm,
             pool_out, tag_out, reu_out, unc_out, inst_out,
             pv, rtv, rmv, g_v, f_v, r_v, u_v, tag_v, fm_v, um_v):
    num_chunks = g_hbm.shape[0] // (CH * G)
    wid = lax.axis_index("s") * 2 + lax.axis_index("c")

    pltpu.sync_copy(p_hbm, pv)
    pltpu.sync_copy(rt_hbm, rtv)
    pltpu.sync_copy(rm_hbm, rmv)

    P = [pv[pl.ds(i * L, L)] for i in range(22)]
    (w00, w01, w02, w03, w10, w11, w12, w13, w20, w21, w22, w23,
     fx, fy, cx0, cy0, n0, n1, n2, f0, f1, f2) = P

    n_my = (num_chunks - wid + NW - 1) // NW

    def chunk_body(i, _):
        c = wid + i * NW
        base = c * CH
        pltpu.sync_copy(g_hbm.at[pl.ds(base * G, CH * G)], g_v)
        pltpu.sync_copy(f_hbm.at[pl.ds(base * D, CH * D)], f_v)

        def mask_body(j, _):
            m26 = rmv[pl.ds(j * L, L)]
            x = plsc.load_gather(g_v, [m26])
            y = plsc.load_gather(g_v, [m26 + 1])
            z = plsc.load_gather(g_v, [m26 + 2])
            def bf16_round(v):
                u = plsc.bitcast(v, jnp.int32)
                r = u + (jnp.int32(0x7FFF) + ((u >> 16) & 1))
                return plsc.bitcast(r & jnp.int32(-65536), jnp.float32)
            xb = bf16_round(x)
            yb = bf16_round(y)
            zb = bf16_round(z)
            cx = xb * w00 + yb * w01 + zb * w02 + w03
            cy = xb * w10 + yb * w11 + zb * w12 + w13
            cz_ = xb * w20 + yb * w21 + zb * w22 + w23
            mask1 = cz_ > 1e-6
            cz = jnp.maximum(cz_, 1e-6)
            pxf = fx * (cx / cz) + cx0
            pyf = fy * (cy / cz) + cy0
            mask2 = ((pxf >= 0.0) & (pxf < 640.0)
                     & (pyf >= 0.0) & (pyf < 480.0))
            in_vox = ((x > n0) & (x < f0) & (y > n1) & (y < f1)
                      & (z > n2) & (z < f2))
            mask_det = mask1 & mask2 & in_vox
            g23 = plsc.load_gather(g_v, [m26 + 23])
            g24 = plsc.load_gather(g_v, [m26 + 24])
            one = jnp.full((L,), 1.0, jnp.float32)
            new24 = jnp.where(in_vox, one, g24)
            plsc.store_scatter(g_v, [m26 + 24], new24)
            half = jnp.full((L,), 0.5, jnp.float32)
            zero = jnp.zeros((L,), jnp.float32)
            tag = jnp.where(mask_det, jnp.where(g23 == 1.0, half, zero), one)
            tag_v[pl.ds(j * L, L)] = tag
            fm_v[pl.ds(j * L, L)] = jnp.where(in_vox, one, zero)
            um_v[pl.ds(j * L, L)] = jnp.where(mask_det, zero, one)
            return 0

        lax.fori_loop(0, NGRP, mask_body, 0)

        def g_body(k, _):
            off = k * L
            row = rtv[pl.ds(off, L)]
            fmv = plsc.load_gather(fm_v, [row])
            umv = plsc.load_gather(um_v, [row])
            gv = g_v[pl.ds(off, L)]
            r_v[pl.ds(off, L)] = gv * fmv
            u_v[pl.ds(off, L)] = gv * umv
            return 0

        lax.fori_loop(0, CH * G // L, g_body, 0)

        def i_body(r, _):
            bidx = jnp.broadcast_to(r, (L,)).astype(jnp.int32)
            fmb = plsc.load_gather(fm_v, [bidx])
            roff = r * D
            for cg in range(D // L):
                off = roff + cg * L
                f_v[pl.ds(off, L)] = f_v[pl.ds(off, L)] * fmb
            return 0

        lax.fori_loop(0, CH, i_body, 0)

        pltpu.sync_copy(g_v, pool_out.at[pl.ds(base * G, CH * G)])
        pltpu.sync_copy(r_v, reu_out.at[pl.ds(base * G, CH * G)])
        pltpu.sync_copy(u_v, unc_out.at[pl.ds(base * G, CH * G)])
        pltpu.sync_copy(f_v, inst_out.at[pl.ds(base * D, CH * D)])
        pltpu.sync_copy(tag_v, tag_out.at[pl.ds(base, CH)])
        return 0

    lax.fori_loop(0, n_my, chunk_body, 0)


def kernel(gaussian_pool, instance_feature_pool, world2cam, cam_k,
           vox_origin, scene_size, mlvl_img_feat, anchor):
    M = gaussian_pool.shape[0]
    batch_size = mlvl_img_feat.shape[0]
    eps = jnp.float32(0.001)
    near = vox_origin + eps
    far = vox_origin + scene_size - eps
    w_rounded = world2cam[:3, :].astype(jnp.bfloat16).astype(jnp.float32)
    params = jnp.concatenate([
        w_rounded.reshape(-1),
        jnp.stack([cam_k[0, 0], cam_k[1, 1], cam_k[0, 2], cam_k[1, 2]]),
        near, far,
    ]).astype(jnp.float32)
    params_b = jnp.repeat(params, L)

    rowtab = jnp.repeat(jnp.arange(CH, dtype=jnp.int32), G)
    rowm26 = jnp.arange(CH, dtype=jnp.int32) * G

    g_flat = gaussian_pool.reshape(-1)
    f_flat = instance_feature_pool.reshape(-1)

    mesh = plsc.VectorSubcoreMesh(core_axis_name="c", subcore_axis_name="s")
    f32 = jnp.float32
    call = pl.kernel(
        _sc_body,
        out_type=[
            jax.ShapeDtypeStruct((M * G,), f32),
            jax.ShapeDtypeStruct((M,), f32),
            jax.ShapeDtypeStruct((M * G,), f32),
            jax.ShapeDtypeStruct((M * G,), f32),
            jax.ShapeDtypeStruct((M * D,), f32),
        ],
        mesh=mesh,
        compiler_params=pltpu.CompilerParams(needs_layout_passes=False),
        scratch_types=[
            pltpu.VMEM((22 * L,), f32),
            pltpu.VMEM((CH * G,), jnp.int32),
            pltpu.VMEM((CH,), jnp.int32),
            pltpu.VMEM((CH * G,), f32),
            pltpu.VMEM((CH * D,), f32),
            pltpu.VMEM((CH * G,), f32),
            pltpu.VMEM((CH * G,), f32),
            pltpu.VMEM((CH,), f32),
            pltpu.VMEM((CH,), f32),
            pltpu.VMEM((CH,), f32),
        ],
    )
    pool_u, tag, reu, unc, inst = call(g_flat, f_flat, params_b, rowtab, rowm26)

    pool_updated = pool_u.reshape(M, G)
    gaussian_reused = reu.reshape(M, G)
    gaussian_unchange = unc.reshape(M, G)
    instance_feature_reused = inst.reshape(M, D)
    anchor_tiled = jnp.tile(anchor[None], (batch_size, 1, 1))
    return (pool_updated, tag, gaussian_reused, gaussian_unchange,
            instance_feature_reused, anchor_tiled)
